# baseline (speedup 1.0000x reference)
import jax
import jax.numpy as jnp
from jax import lax
from jax.experimental import pallas as pl
from jax.experimental.pallas import tpu as pltpu
from jax.experimental.pallas import tpu_sc as plsc

NC = 2
NS = 16
CHUNK = 128
ZROWS = 128
ALPHA = 0.1
BN = 640


def _copy_row(src2d, k, dst1d, off):
    for j in range(CHUNK // 16):
        dst1d[pl.ds(16 * j, 16)] = src2d[k, pl.ds(16 * j, 16)] + off


def _mesh():
    return plsc.VectorSubcoreMesh(
        core_axis_name="c", subcore_axis_name="s", num_cores=NC, num_subcores=NS
    )


def _sc_degree(dst3d, npad):
    ncha = dst3d.shape[1]
    rpt = npad // NS

    def body(dst_hbm, deg_hbm, dstv, di_a, di_b, ones, zb, acc, ssem_a, ssem_b):
        c = lax.axis_index("c")
        s = lax.axis_index("s")
        w = c * NS + s
        pltpu.sync_copy(dst_hbm.at[w], dstv)

        def fill_ones(i, carry):
            for j in range(8):
                ones[i, pl.ds(16 * j, 16)] = jnp.ones((16,), jnp.float32)
            return carry

        lax.fori_loop(0, CHUNK, fill_ones, 0)

        def fill_z(i, carry):
            for j in range(8):
                zb[i, pl.ds(16 * j, 16)] = jnp.zeros((16,), jnp.float32)
            return carry

        lax.fori_loop(0, ZROWS, fill_z, 0)
        row0 = s * rpt
        for k in range(rpt // ZROWS):
            pltpu.sync_copy(zb, acc.at[pl.ds(row0 + k * ZROWS, ZROWS)])
        plsc.subcore_barrier()

        _copy_row(dstv, 0, di_a, 0)
        pltpu.async_copy(ones, acc.at[di_a], ssem_a, add=True)

        def add_pair(i, carry):
            k1 = 2 * i + 1
            k2 = 2 * i + 2
            _copy_row(dstv, k1, di_b, 0)
            pltpu.async_copy(ones, acc.at[di_b], ssem_b, add=True)
            pltpu.make_async_copy(ones, acc.at[di_a], ssem_a).wait()

            @pl.when(k2 < ncha)
            def _():
                _copy_row(dstv, k2, di_a, 0)
                pltpu.async_copy(ones, acc.at[di_a], ssem_a, add=True)

            pltpu.make_async_copy(ones, acc.at[di_b], ssem_b).wait()
            return carry

        lax.fori_loop(0, ncha // 2, add_pair, 0)
        plsc.subcore_barrier()
        pltpu.sync_copy(
            acc.at[pl.ds(row0, rpt)], deg_hbm.at[pl.ds(c * npad + row0, rpt)]
        )

    f = pl.kernel(
        body,
        out_type=[jax.ShapeDtypeStruct((NC * npad, 128), jnp.float32)],
        mesh=_mesh(),
        scratch_types=[
            pltpu.VMEM((ncha, CHUNK), jnp.int32),
            pltpu.VMEM((CHUNK,), jnp.int32),
            pltpu.VMEM((CHUNK,), jnp.int32),
            pltpu.VMEM((CHUNK, 128), jnp.float32),
            pltpu.VMEM((ZROWS, 128), jnp.float32),
            pltpu.VMEM_SHARED((npad, 128), jnp.float32),
            pltpu.SemaphoreType.DMA,
            pltpu.SemaphoreType.DMA,
        ],
    )
    (deg,) = f(dst3d)
    return deg


def _sc_aggregate(xs_flat, src3, dst_flat, npad):
    nch = src3.shape[1]
    rpt = npad // NS
    etile = nch * CHUNK

    def body(xs_hbm, src_hbm, dst_hbm, agg_hbm,
             srcv, si_a, si_b, di_a, di_b, buf_a, buf_b, acc,
             gsem_a, gsem_b, dsem_a, dsem_b):
        c = lax.axis_index("c")
        s = lax.axis_index("s")
        off = c * npad
        base = s * etile
        pltpu.sync_copy(src_hbm.at[s], srcv)

        def fill_z(i, carry):
            for j in range(8):
                buf_a[i, pl.ds(16 * j, 16)] = jnp.zeros((16,), jnp.float32)
            return carry

        lax.fori_loop(0, ZROWS, fill_z, 0)
        row0 = s * rpt
        for k in range(rpt // ZROWS):
            pltpu.sync_copy(
                buf_a.at[pl.ds(0, ZROWS)], acc.at[pl.ds(row0 + k * ZROWS, ZROWS)]
            )
        plsc.subcore_barrier()

        def fetch_di(k, di, dsem):
            pltpu.async_copy(dst_hbm.at[pl.ds(base + k * CHUNK, CHUNK)], di, dsem)

        def wait_di(k, di, dsem):
            pltpu.make_async_copy(
                dst_hbm.at[pl.ds(base + k * CHUNK, CHUNK)], di, dsem
            ).wait()

        _copy_row(srcv, 0, si_a, off)
        pltpu.async_copy(xs_hbm.at[si_a], buf_a, gsem_a)
        fetch_di(0, di_a, dsem_a)

        def step(i, carry):
            k0 = 2 * i
            k1 = 2 * i + 1
            k2 = 2 * i + 2
            _copy_row(srcv, k1, si_b, off)
            pltpu.async_copy(xs_hbm.at[si_b], buf_b, gsem_b)
            fetch_di(k1, di_b, dsem_b)
            pltpu.make_async_copy(xs_hbm.at[si_a], buf_a, gsem_a).wait()
            wait_di(k0, di_a, dsem_a)
            pltpu.sync_copy(buf_a, acc.at[di_a], add=True)

            @pl.when(k2 < nch)
            def _():
                _copy_row(srcv, k2, si_a, off)
                pltpu.async_copy(xs_hbm.at[si_a], buf_a, gsem_a)
                fetch_di(k2, di_a, dsem_a)

            pltpu.make_async_copy(xs_hbm.at[si_b], buf_b, gsem_b).wait()
            wait_di(k1, di_b, dsem_b)
            pltpu.sync_copy(buf_b, acc.at[di_b], add=True)
            return carry

        lax.fori_loop(0, nch // 2, step, 0)
        plsc.subcore_barrier()
        pltpu.sync_copy(
            acc.at[pl.ds(row0, rpt)], agg_hbm.at[pl.ds(off + row0, rpt)]
        )

    f = pl.kernel(
        body,
        out_type=[jax.ShapeDtypeStruct((NC * npad, 128), jnp.float32)],
        mesh=_mesh(),
        scratch_types=[
            pltpu.VMEM((nch, CHUNK), jnp.int32),
            pltpu.VMEM((CHUNK,), jnp.int32),
            pltpu.VMEM((CHUNK,), jnp.int32),
            pltpu.VMEM((CHUNK,), jnp.int32),
            pltpu.VMEM((CHUNK,), jnp.int32),
            pltpu.VMEM((CHUNK, 128), jnp.float32),
            pltpu.VMEM((CHUNK, 128), jnp.float32),
            pltpu.VMEM_SHARED((npad, 128), jnp.float32),
            pltpu.SemaphoreType.DMA,
            pltpu.SemaphoreType.DMA,
            pltpu.SemaphoreType.DMA,
            pltpu.SemaphoreType.DMA,
        ],
    )
    (agg,) = f(xs_flat, src3, dst_flat)
    return agg


def _tc_prep(x_pad, deg3, npad):
    def body(x_ref, d_ref, r_ref, xs_ref):
        d3 = d_ref[...]
        d = d3[0, :, :1] + d3[1, :, :1]
        r = lax.rsqrt(jnp.maximum(d, 1.0))
        r_ref[...] = jnp.broadcast_to(r, (BN, 16))
        xs = x_ref[...] * r
        xs_ref[0] = xs[:, :128]
        xs_ref[1] = xs[:, 128:]

    grid = (npad // BN,)
    return pl.pallas_call(
        body,
        grid=grid,
        in_specs=[
            pl.BlockSpec((BN, 256), lambda i: (i, 0)),
            pl.BlockSpec((2, BN, 128), lambda i: (0, i, 0)),
        ],
        out_specs=[
            pl.BlockSpec((BN, 16), lambda i: (i, 0)),
            pl.BlockSpec((2, BN, 128), lambda i: (0, i, 0)),
        ],
        out_shape=[
            jax.ShapeDtypeStruct((npad, 16), jnp.float32),
            jax.ShapeDtypeStruct((2, npad, 128), jnp.float32),
        ],
    )(x_pad, deg3)


def _tc_layer(agg3, x_pad, r16, w, n, npad, emit_next):
    def body(a_ref, x_ref, r_ref, w_ref, o_ref, *next_refs):
        r1 = r_ref[..., :1]
        a3 = a_ref[...]
        agg = jnp.concatenate([a3[0], a3[1]], axis=1)
        hi = (1.0 - ALPHA) * (r1 * agg) + ALPHA * x_ref[...]
        o = jnp.maximum(jnp.dot(hi, w_ref[...], preferred_element_type=jnp.float32), 0.0)
        o_ref[...] = o
        if emit_next:
            xs = o * r1
            next_refs[0][0] = xs[:, :128]
            next_refs[0][1] = xs[:, 128:]

    grid = (npad // BN,)
    out_specs = [pl.BlockSpec((BN, 256), lambda i: (i, 0))]
    out_shape = [jax.ShapeDtypeStruct((n, 256), jnp.float32)]
    if emit_next:
        out_specs.append(pl.BlockSpec((2, BN, 128), lambda i: (0, i, 0)))
        out_shape.append(jax.ShapeDtypeStruct((2, npad, 128), jnp.float32))
    return pl.pallas_call(
        body,
        grid=grid,
        in_specs=[
            pl.BlockSpec((2, BN, 128), lambda i: (0, i, 0)),
            pl.BlockSpec((BN, 256), lambda i: (i, 0)),
            pl.BlockSpec((BN, 16), lambda i: (i, 0)),
            pl.BlockSpec((256, 256), lambda i: (0, 0)),
        ],
        out_specs=out_specs,
        out_shape=out_shape,
    )(agg3, x_pad, r16, w)


def kernel(drug_cell_pair_feature, edge_idx, W):
    x = drug_cell_pair_feature
    n, d = x.shape
    e = edge_idx.shape[1]
    assert d == 256

    npad = ((n + 1 + NS * ZROWS - 1) // (NS * ZROWS)) * (NS * ZROWS)
    epad = ((e + 2 * NS * CHUNK - 1) // (2 * NS * CHUNK)) * (2 * NS * CHUNK)
    nch = epad // (NS * CHUNK)

    src = edge_idx[0]
    dst = edge_idx[1]
    src_pad = jnp.concatenate([src, jnp.zeros((epad - e,), jnp.int32)])
    dst_pad = jnp.concatenate([dst, jnp.full((epad - e,), n, jnp.int32)])
    src3 = src_pad.reshape(NS, nch, CHUNK)
    dst3d = dst_pad.reshape(NC * NS, nch // NC, CHUNK)
    x_pad = jnp.concatenate([x, jnp.zeros((npad - n, d), jnp.float32)])

    deg = _sc_degree(dst3d, npad)
    r16, xs3 = _tc_prep(x_pad, deg.reshape(NC, npad, 128), npad)

    outs = []
    cur = xs3
    for layer in range(5):
        agg = _sc_aggregate(cur.reshape(NC * npad, 128), src3, dst_pad, npad)
        agg3 = agg.reshape(NC, npad, 128)
        if layer < 4:
            o, cur = _tc_layer(agg3, x_pad, r16, W, n, npad, True)
        else:
            (o,) = _tc_layer(agg3, x_pad, r16, W, n, npad, False)
        outs.append(o)
    return tuple(outs)


# speedup vs baseline: 1.0142x; 1.0142x over previous
import jax
import jax.numpy as jnp
from jax import lax
from jax.experimental import pallas as pl
from jax.experimental.pallas import tpu as pltpu
from jax.experimental.pallas import tpu_sc as plsc

NC = 2
NS = 16
CHUNK = 128
ZROWS = 128
ALPHA = 0.1
BN = 1024


def _copy_row(src2d, k, dst1d, off):
    for j in range(CHUNK // 16):
        dst1d[pl.ds(16 * j, 16)] = src2d[k, pl.ds(16 * j, 16)] + off


def _mesh():
    return plsc.VectorSubcoreMesh(
        core_axis_name="c", subcore_axis_name="s", num_cores=NC, num_subcores=NS
    )


def _sc_degree(dst3d, npad):
    ncha = dst3d.shape[1]
    rpt = npad // NS

    def body(dst_hbm, deg_hbm, dstv, di_a, di_b, ones, zb, acc, ssem_a, ssem_b):
        c = lax.axis_index("c")
        s = lax.axis_index("s")
        w = c * NS + s
        pltpu.sync_copy(dst_hbm.at[w], dstv)

        def fill_ones(i, carry):
            for j in range(8):
                ones[i, pl.ds(16 * j, 16)] = jnp.ones((16,), jnp.float32)
            return carry

        lax.fori_loop(0, CHUNK, fill_ones, 0)

        def fill_z(i, carry):
            for j in range(8):
                zb[i, pl.ds(16 * j, 16)] = jnp.zeros((16,), jnp.float32)
            return carry

        lax.fori_loop(0, ZROWS, fill_z, 0)
        row0 = s * rpt
        for k in range(rpt // ZROWS):
            pltpu.sync_copy(zb, acc.at[pl.ds(row0 + k * ZROWS, ZROWS)])
        plsc.subcore_barrier()

        _copy_row(dstv, 0, di_a, 0)
        pltpu.async_copy(ones, acc.at[di_a], ssem_a, add=True)

        def add_pair(i, carry):
            k1 = 2 * i + 1
            k2 = 2 * i + 2
            _copy_row(dstv, k1, di_b, 0)
            pltpu.async_copy(ones, acc.at[di_b], ssem_b, add=True)
            pltpu.make_async_copy(ones, acc.at[di_a], ssem_a).wait()

            @pl.when(k2 < ncha)
            def _():
                _copy_row(dstv, k2, di_a, 0)
                pltpu.async_copy(ones, acc.at[di_a], ssem_a, add=True)

            pltpu.make_async_copy(ones, acc.at[di_b], ssem_b).wait()
            return carry

        lax.fori_loop(0, ncha // 2, add_pair, 0)
        plsc.subcore_barrier()
        pltpu.sync_copy(
            acc.at[pl.ds(row0, rpt)], deg_hbm.at[pl.ds(c * npad + row0, rpt)]
        )

    f = pl.kernel(
        body,
        out_type=[jax.ShapeDtypeStruct((NC * npad, 128), jnp.float32)],
        mesh=_mesh(),
        scratch_types=[
            pltpu.VMEM((ncha, CHUNK), jnp.int32),
            pltpu.VMEM((CHUNK,), jnp.int32),
            pltpu.VMEM((CHUNK,), jnp.int32),
            pltpu.VMEM((CHUNK, 128), jnp.float32),
            pltpu.VMEM((ZROWS, 128), jnp.float32),
            pltpu.VMEM_SHARED((npad, 128), jnp.float32),
            pltpu.SemaphoreType.DMA,
            pltpu.SemaphoreType.DMA,
        ],
    )
    (deg,) = f(dst3d)
    return deg


def _sc_aggregate(xs_flat, src3, dst_flat, npad):
    nch = src3.shape[1]
    rpt = npad // NS
    etile = nch * CHUNK

    def body(xs_hbm, src_hbm, dst_hbm, agg_hbm,
             srcv, si_a, si_b, di_a, di_b, buf_a, buf_b, acc,
             gsem_a, gsem_b, dsem_a, dsem_b):
        c = lax.axis_index("c")
        s = lax.axis_index("s")
        off = c * npad
        base = s * etile
        pltpu.sync_copy(src_hbm.at[s], srcv)

        def fill_z(i, carry):
            for j in range(8):
                buf_a[i, pl.ds(16 * j, 16)] = jnp.zeros((16,), jnp.float32)
            return carry

        lax.fori_loop(0, ZROWS, fill_z, 0)
        row0 = s * rpt
        for k in range(rpt // ZROWS):
            pltpu.sync_copy(
                buf_a.at[pl.ds(0, ZROWS)], acc.at[pl.ds(row0 + k * ZROWS, ZROWS)]
            )
        plsc.subcore_barrier()

        def fetch_di(k, di, dsem):
            pltpu.async_copy(dst_hbm.at[pl.ds(base + k * CHUNK, CHUNK)], di, dsem)

        def wait_di(k, di, dsem):
            pltpu.make_async_copy(
                dst_hbm.at[pl.ds(base + k * CHUNK, CHUNK)], di, dsem
            ).wait()

        _copy_row(srcv, 0, si_a, off)
        pltpu.async_copy(xs_hbm.at[si_a], buf_a, gsem_a)
        fetch_di(0, di_a, dsem_a)

        def step(i, carry):
            k0 = 2 * i
            k1 = 2 * i + 1
            k2 = 2 * i + 2
            _copy_row(srcv, k1, si_b, off)
            pltpu.async_copy(xs_hbm.at[si_b], buf_b, gsem_b)
            fetch_di(k1, di_b, dsem_b)
            pltpu.make_async_copy(xs_hbm.at[si_a], buf_a, gsem_a).wait()
            wait_di(k0, di_a, dsem_a)
            pltpu.sync_copy(buf_a, acc.at[di_a], add=True)

            @pl.when(k2 < nch)
            def _():
                _copy_row(srcv, k2, si_a, off)
                pltpu.async_copy(xs_hbm.at[si_a], buf_a, gsem_a)
                fetch_di(k2, di_a, dsem_a)

            pltpu.make_async_copy(xs_hbm.at[si_b], buf_b, gsem_b).wait()
            wait_di(k1, di_b, dsem_b)
            pltpu.sync_copy(buf_b, acc.at[di_b], add=True)
            return carry

        lax.fori_loop(0, nch // 2, step, 0)
        plsc.subcore_barrier()
        pltpu.sync_copy(
            acc.at[pl.ds(row0, rpt)], agg_hbm.at[pl.ds(off + row0, rpt)]
        )

    f = pl.kernel(
        body,
        out_type=[jax.ShapeDtypeStruct((NC * npad, 128), jnp.float32)],
        mesh=_mesh(),
        scratch_types=[
            pltpu.VMEM((nch, CHUNK), jnp.int32),
            pltpu.VMEM((CHUNK,), jnp.int32),
            pltpu.VMEM((CHUNK,), jnp.int32),
            pltpu.VMEM((CHUNK,), jnp.int32),
            pltpu.VMEM((CHUNK,), jnp.int32),
            pltpu.VMEM((CHUNK, 128), jnp.float32),
            pltpu.VMEM((CHUNK, 128), jnp.float32),
            pltpu.VMEM_SHARED((npad, 128), jnp.float32),
            pltpu.SemaphoreType.DMA,
            pltpu.SemaphoreType.DMA,
            pltpu.SemaphoreType.DMA,
            pltpu.SemaphoreType.DMA,
        ],
    )
    (agg,) = f(xs_flat, src3, dst_flat)
    return agg


def _tc_prep(x_pad, deg3, npad):
    def body(x_ref, d_ref, r_ref, xs_ref):
        d3 = d_ref[...]
        d = d3[0, :, :1] + d3[1, :, :1]
        r = lax.rsqrt(jnp.maximum(d, 1.0))
        r_ref[...] = jnp.broadcast_to(r, (BN, 16))
        xs = x_ref[...] * r
        xs_ref[0] = xs[:, :128]
        xs_ref[1] = xs[:, 128:]

    grid = (npad // BN,)
    return pl.pallas_call(
        body,
        grid=grid,
        in_specs=[
            pl.BlockSpec((BN, 256), lambda i: (i, 0)),
            pl.BlockSpec((2, BN, 128), lambda i: (0, i, 0)),
        ],
        out_specs=[
            pl.BlockSpec((BN, 16), lambda i: (i, 0)),
            pl.BlockSpec((2, BN, 128), lambda i: (0, i, 0)),
        ],
        out_shape=[
            jax.ShapeDtypeStruct((npad, 16), jnp.float32),
            jax.ShapeDtypeStruct((2, npad, 128), jnp.float32),
        ],
    )(x_pad, deg3)


def _tc_layer(agg3, x_pad, r16, w, n, npad, emit_next):
    def body(a_ref, x_ref, r_ref, w_ref, o_ref, *next_refs):
        r1 = r_ref[..., :1]
        a3 = a_ref[...]
        agg = jnp.concatenate([a3[0], a3[1]], axis=1)
        hi = (1.0 - ALPHA) * (r1 * agg) + ALPHA * x_ref[...]
        o = jnp.maximum(jnp.dot(hi, w_ref[...], preferred_element_type=jnp.float32), 0.0)
        o_ref[...] = o
        if emit_next:
            xs = o * r1
            next_refs[0][0] = xs[:, :128]
            next_refs[0][1] = xs[:, 128:]

    grid = (npad // BN,)
    out_specs = [pl.BlockSpec((BN, 256), lambda i: (i, 0))]
    out_shape = [jax.ShapeDtypeStruct((n, 256), jnp.float32)]
    if emit_next:
        out_specs.append(pl.BlockSpec((2, BN, 128), lambda i: (0, i, 0)))
        out_shape.append(jax.ShapeDtypeStruct((2, npad, 128), jnp.float32))
    return pl.pallas_call(
        body,
        grid=grid,
        in_specs=[
            pl.BlockSpec((2, BN, 128), lambda i: (0, i, 0)),
            pl.BlockSpec((BN, 256), lambda i: (i, 0)),
            pl.BlockSpec((BN, 16), lambda i: (i, 0)),
            pl.BlockSpec((256, 256), lambda i: (0, 0)),
        ],
        out_specs=out_specs,
        out_shape=out_shape,
    )(agg3, x_pad, r16, w)


def kernel(drug_cell_pair_feature, edge_idx, W):
    x = drug_cell_pair_feature
    n, d = x.shape
    e = edge_idx.shape[1]
    assert d == 256

    npad = ((n + 1 + NS * ZROWS - 1) // (NS * ZROWS)) * (NS * ZROWS)
    epad = ((e + 2 * NS * CHUNK - 1) // (2 * NS * CHUNK)) * (2 * NS * CHUNK)
    nch = epad // (NS * CHUNK)

    src = edge_idx[0]
    dst = edge_idx[1]
    src_pad = jnp.concatenate([src, jnp.zeros((epad - e,), jnp.int32)])
    dst_pad = jnp.concatenate([dst, jnp.full((epad - e,), n, jnp.int32)])
    src3 = src_pad.reshape(NS, nch, CHUNK)
    dst3d = dst_pad.reshape(NC * NS, nch // NC, CHUNK)
    x_pad = jnp.concatenate([x, jnp.zeros((npad - n, d), jnp.float32)])

    deg = _sc_degree(dst3d, npad)
    r16, xs3 = _tc_prep(x_pad, deg.reshape(NC, npad, 128), npad)

    outs = []
    cur = xs3
    for layer in range(5):
        agg = _sc_aggregate(cur.reshape(NC * npad, 128), src3, dst_pad, npad)
        agg3 = agg.reshape(NC, npad, 128)
        if layer < 4:
            o, cur = _tc_layer(agg3, x_pad, r16, W, n, npad, True)
        else:
            (o,) = _tc_layer(agg3, x_pad, r16, W, n, npad, False)
        outs.append(o)
    return tuple(outs)


# speedup vs baseline: 1.0191x; 1.0048x over previous
import jax
import jax.numpy as jnp
from jax import lax
from jax.experimental import pallas as pl
from jax.experimental.pallas import tpu as pltpu
from jax.experimental.pallas import tpu_sc as plsc

NC = 2
NS = 16
CHUNK = 128
ZROWS = 128
ALPHA = 0.1
BN = 2048


def _copy_row(src2d, k, dst1d, off):
    for j in range(CHUNK // 16):
        dst1d[pl.ds(16 * j, 16)] = src2d[k, pl.ds(16 * j, 16)] + off


def _mesh():
    return plsc.VectorSubcoreMesh(
        core_axis_name="c", subcore_axis_name="s", num_cores=NC, num_subcores=NS
    )


def _sc_degree(dst3d, npad):
    ncha = dst3d.shape[1]
    rpt = npad // NS

    def body(dst_hbm, deg_hbm, dstv, di_a, di_b, ones, zb, acc, ssem_a, ssem_b):
        c = lax.axis_index("c")
        s = lax.axis_index("s")
        w = c * NS + s
        pltpu.sync_copy(dst_hbm.at[w], dstv)

        def fill_ones(i, carry):
            for j in range(8):
                ones[i, pl.ds(16 * j, 16)] = jnp.ones((16,), jnp.float32)
            return carry

        lax.fori_loop(0, CHUNK, fill_ones, 0)

        def fill_z(i, carry):
            for j in range(8):
                zb[i, pl.ds(16 * j, 16)] = jnp.zeros((16,), jnp.float32)
            return carry

        lax.fori_loop(0, ZROWS, fill_z, 0)
        row0 = s * rpt
        for k in range(rpt // ZROWS):
            pltpu.sync_copy(zb, acc.at[pl.ds(row0 + k * ZROWS, ZROWS)])
        plsc.subcore_barrier()

        _copy_row(dstv, 0, di_a, 0)
        pltpu.async_copy(ones, acc.at[di_a], ssem_a, add=True)

        def add_pair(i, carry):
            k1 = 2 * i + 1
            k2 = 2 * i + 2
            _copy_row(dstv, k1, di_b, 0)
            pltpu.async_copy(ones, acc.at[di_b], ssem_b, add=True)
            pltpu.make_async_copy(ones, acc.at[di_a], ssem_a).wait()

            @pl.when(k2 < ncha)
            def _():
                _copy_row(dstv, k2, di_a, 0)
                pltpu.async_copy(ones, acc.at[di_a], ssem_a, add=True)

            pltpu.make_async_copy(ones, acc.at[di_b], ssem_b).wait()
            return carry

        lax.fori_loop(0, ncha // 2, add_pair, 0)
        plsc.subcore_barrier()
        pltpu.sync_copy(
            acc.at[pl.ds(row0, rpt)], deg_hbm.at[pl.ds(c * npad + row0, rpt)]
        )

    f = pl.kernel(
        body,
        out_type=[jax.ShapeDtypeStruct((NC * npad, 128), jnp.float32)],
        mesh=_mesh(),
        scratch_types=[
            pltpu.VMEM((ncha, CHUNK), jnp.int32),
            pltpu.VMEM((CHUNK,), jnp.int32),
            pltpu.VMEM((CHUNK,), jnp.int32),
            pltpu.VMEM((CHUNK, 128), jnp.float32),
            pltpu.VMEM((ZROWS, 128), jnp.float32),
            pltpu.VMEM_SHARED((npad, 128), jnp.float32),
            pltpu.SemaphoreType.DMA,
            pltpu.SemaphoreType.DMA,
        ],
    )
    (deg,) = f(dst3d)
    return deg


def _sc_aggregate(xs_flat, src3, dst_flat, npad):
    nch = src3.shape[1]
    rpt = npad // NS
    etile = nch * CHUNK

    def body(xs_hbm, src_hbm, dst_hbm, agg_hbm,
             srcv, si_a, si_b, di_a, di_b, buf_a, buf_b, acc,
             gsem_a, gsem_b, dsem_a, dsem_b):
        c = lax.axis_index("c")
        s = lax.axis_index("s")
        off = c * npad
        base = s * etile
        pltpu.sync_copy(src_hbm.at[s], srcv)

        def fill_z(i, carry):
            for j in range(8):
                buf_a[i, pl.ds(16 * j, 16)] = jnp.zeros((16,), jnp.float32)
            return carry

        lax.fori_loop(0, ZROWS, fill_z, 0)
        row0 = s * rpt
        for k in range(rpt // ZROWS):
            pltpu.sync_copy(
                buf_a.at[pl.ds(0, ZROWS)], acc.at[pl.ds(row0 + k * ZROWS, ZROWS)]
            )
        plsc.subcore_barrier()

        def fetch_di(k, di, dsem):
            pltpu.async_copy(dst_hbm.at[pl.ds(base + k * CHUNK, CHUNK)], di, dsem)

        def wait_di(k, di, dsem):
            pltpu.make_async_copy(
                dst_hbm.at[pl.ds(base + k * CHUNK, CHUNK)], di, dsem
            ).wait()

        _copy_row(srcv, 0, si_a, off)
        pltpu.async_copy(xs_hbm.at[si_a], buf_a, gsem_a)
        fetch_di(0, di_a, dsem_a)

        def step(i, carry):
            k0 = 2 * i
            k1 = 2 * i + 1
            k2 = 2 * i + 2
            _copy_row(srcv, k1, si_b, off)
            pltpu.async_copy(xs_hbm.at[si_b], buf_b, gsem_b)
            fetch_di(k1, di_b, dsem_b)
            pltpu.make_async_copy(xs_hbm.at[si_a], buf_a, gsem_a).wait()
            wait_di(k0, di_a, dsem_a)
            pltpu.sync_copy(buf_a, acc.at[di_a], add=True)

            @pl.when(k2 < nch)
            def _():
                _copy_row(srcv, k2, si_a, off)
                pltpu.async_copy(xs_hbm.at[si_a], buf_a, gsem_a)
                fetch_di(k2, di_a, dsem_a)

            pltpu.make_async_copy(xs_hbm.at[si_b], buf_b, gsem_b).wait()
            wait_di(k1, di_b, dsem_b)
            pltpu.sync_copy(buf_b, acc.at[di_b], add=True)
            return carry

        lax.fori_loop(0, nch // 2, step, 0)
        plsc.subcore_barrier()
        pltpu.sync_copy(
            acc.at[pl.ds(row0, rpt)], agg_hbm.at[pl.ds(off + row0, rpt)]
        )

    f = pl.kernel(
        body,
        out_type=[jax.ShapeDtypeStruct((NC * npad, 128), jnp.float32)],
        mesh=_mesh(),
        scratch_types=[
            pltpu.VMEM((nch, CHUNK), jnp.int32),
            pltpu.VMEM((CHUNK,), jnp.int32),
            pltpu.VMEM((CHUNK,), jnp.int32),
            pltpu.VMEM((CHUNK,), jnp.int32),
            pltpu.VMEM((CHUNK,), jnp.int32),
            pltpu.VMEM((CHUNK, 128), jnp.float32),
            pltpu.VMEM((CHUNK, 128), jnp.float32),
            pltpu.VMEM_SHARED((npad, 128), jnp.float32),
            pltpu.SemaphoreType.DMA,
            pltpu.SemaphoreType.DMA,
            pltpu.SemaphoreType.DMA,
            pltpu.SemaphoreType.DMA,
        ],
    )
    (agg,) = f(xs_flat, src3, dst_flat)
    return agg


def _tc_prep(x_pad, deg3, npad):
    def body(x_ref, d_ref, r_ref, xs_ref):
        d3 = d_ref[...]
        d = d3[0, :, :1] + d3[1, :, :1]
        r = lax.rsqrt(jnp.maximum(d, 1.0))
        r_ref[...] = jnp.broadcast_to(r, (BN, 16))
        xs = x_ref[...] * r
        xs_ref[0] = xs[:, :128]
        xs_ref[1] = xs[:, 128:]

    grid = (npad // BN,)
    return pl.pallas_call(
        body,
        grid=grid,
        in_specs=[
            pl.BlockSpec((BN, 256), lambda i: (i, 0)),
            pl.BlockSpec((2, BN, 128), lambda i: (0, i, 0)),
        ],
        out_specs=[
            pl.BlockSpec((BN, 16), lambda i: (i, 0)),
            pl.BlockSpec((2, BN, 128), lambda i: (0, i, 0)),
        ],
        out_shape=[
            jax.ShapeDtypeStruct((npad, 16), jnp.float32),
            jax.ShapeDtypeStruct((2, npad, 128), jnp.float32),
        ],
    )(x_pad, deg3)


def _tc_layer(agg3, x_pad, r16, w, n, npad, emit_next):
    def body(a_ref, x_ref, r_ref, w_ref, o_ref, *next_refs):
        r1 = r_ref[..., :1]
        a3 = a_ref[...]
        agg = jnp.concatenate([a3[0], a3[1]], axis=1)
        hi = (1.0 - ALPHA) * (r1 * agg) + ALPHA * x_ref[...]
        o = jnp.maximum(jnp.dot(hi, w_ref[...], preferred_element_type=jnp.float32), 0.0)
        o_ref[...] = o
        if emit_next:
            xs = o * r1
            next_refs[0][0] = xs[:, :128]
            next_refs[0][1] = xs[:, 128:]

    grid = (npad // BN,)
    out_specs = [pl.BlockSpec((BN, 256), lambda i: (i, 0))]
    out_shape = [jax.ShapeDtypeStruct((n, 256), jnp.float32)]
    if emit_next:
        out_specs.append(pl.BlockSpec((2, BN, 128), lambda i: (0, i, 0)))
        out_shape.append(jax.ShapeDtypeStruct((2, npad, 128), jnp.float32))
    return pl.pallas_call(
        body,
        grid=grid,
        in_specs=[
            pl.BlockSpec((2, BN, 128), lambda i: (0, i, 0)),
            pl.BlockSpec((BN, 256), lambda i: (i, 0)),
            pl.BlockSpec((BN, 16), lambda i: (i, 0)),
            pl.BlockSpec((256, 256), lambda i: (0, 0)),
        ],
        out_specs=out_specs,
        out_shape=out_shape,
    )(agg3, x_pad, r16, w)


def kernel(drug_cell_pair_feature, edge_idx, W):
    x = drug_cell_pair_feature
    n, d = x.shape
    e = edge_idx.shape[1]
    assert d == 256

    npad = ((n + 1 + NS * ZROWS - 1) // (NS * ZROWS)) * (NS * ZROWS)
    epad = ((e + 2 * NS * CHUNK - 1) // (2 * NS * CHUNK)) * (2 * NS * CHUNK)
    nch = epad // (NS * CHUNK)

    src = edge_idx[0]
    dst = edge_idx[1]
    src_pad = jnp.concatenate([src, jnp.zeros((epad - e,), jnp.int32)])
    dst_pad = jnp.concatenate([dst, jnp.full((epad - e,), n, jnp.int32)])
    src3 = src_pad.reshape(NS, nch, CHUNK)
    dst3d = dst_pad.reshape(NC * NS, nch // NC, CHUNK)
    x_pad = jnp.concatenate([x, jnp.zeros((npad - n, d), jnp.float32)])

    deg = _sc_degree(dst3d, npad)
    r16, xs3 = _tc_prep(x_pad, deg.reshape(NC, npad, 128), npad)

    outs = []
    cur = xs3
    for layer in range(5):
        agg = _sc_aggregate(cur.reshape(NC * npad, 128), src3, dst_pad, npad)
        agg3 = agg.reshape(NC, npad, 128)
        if layer < 4:
            o, cur = _tc_layer(agg3, x_pad, r16, W, n, npad, True)
        else:
            (o,) = _tc_layer(agg3, x_pad, r16, W, n, npad, False)
        outs.append(o)
    return tuple(outs)


# speedup vs baseline: 1.0243x; 1.0051x over previous
import jax
import jax.numpy as jnp
from jax import lax
from jax.experimental import pallas as pl
from jax.experimental.pallas import tpu as pltpu
from jax.experimental.pallas import tpu_sc as plsc

NC = 2
NS = 16
CHUNK = 128
ZROWS = 128
ALPHA = 0.1
BN = 5120


def _copy_row(src2d, k, dst1d, off):
    for j in range(CHUNK // 16):
        dst1d[pl.ds(16 * j, 16)] = src2d[k, pl.ds(16 * j, 16)] + off


def _mesh():
    return plsc.VectorSubcoreMesh(
        core_axis_name="c", subcore_axis_name="s", num_cores=NC, num_subcores=NS
    )


def _sc_degree(dst3d, npad):
    ncha = dst3d.shape[1]
    rpt = npad // NS

    def body(dst_hbm, deg_hbm, dstv, di_a, di_b, ones, zb, acc, ssem_a, ssem_b):
        c = lax.axis_index("c")
        s = lax.axis_index("s")
        w = c * NS + s
        pltpu.sync_copy(dst_hbm.at[w], dstv)

        def fill_ones(i, carry):
            for j in range(8):
                ones[i, pl.ds(16 * j, 16)] = jnp.ones((16,), jnp.float32)
            return carry

        lax.fori_loop(0, CHUNK, fill_ones, 0)

        def fill_z(i, carry):
            for j in range(8):
                zb[i, pl.ds(16 * j, 16)] = jnp.zeros((16,), jnp.float32)
            return carry

        lax.fori_loop(0, ZROWS, fill_z, 0)
        row0 = s * rpt
        for k in range(rpt // ZROWS):
            pltpu.sync_copy(zb, acc.at[pl.ds(row0 + k * ZROWS, ZROWS)])
        plsc.subcore_barrier()

        _copy_row(dstv, 0, di_a, 0)
        pltpu.async_copy(ones, acc.at[di_a], ssem_a, add=True)

        def add_pair(i, carry):
            k1 = 2 * i + 1
            k2 = 2 * i + 2
            _copy_row(dstv, k1, di_b, 0)
            pltpu.async_copy(ones, acc.at[di_b], ssem_b, add=True)
            pltpu.make_async_copy(ones, acc.at[di_a], ssem_a).wait()

            @pl.when(k2 < ncha)
            def _():
                _copy_row(dstv, k2, di_a, 0)
                pltpu.async_copy(ones, acc.at[di_a], ssem_a, add=True)

            pltpu.make_async_copy(ones, acc.at[di_b], ssem_b).wait()
            return carry

        lax.fori_loop(0, ncha // 2, add_pair, 0)
        plsc.subcore_barrier()
        pltpu.sync_copy(
            acc.at[pl.ds(row0, rpt)], deg_hbm.at[pl.ds(c * npad + row0, rpt)]
        )

    f = pl.kernel(
        body,
        out_type=[jax.ShapeDtypeStruct((NC * npad, 128), jnp.float32)],
        mesh=_mesh(),
        scratch_types=[
            pltpu.VMEM((ncha, CHUNK), jnp.int32),
            pltpu.VMEM((CHUNK,), jnp.int32),
            pltpu.VMEM((CHUNK,), jnp.int32),
            pltpu.VMEM((CHUNK, 128), jnp.float32),
            pltpu.VMEM((ZROWS, 128), jnp.float32),
            pltpu.VMEM_SHARED((npad, 128), jnp.float32),
            pltpu.SemaphoreType.DMA,
            pltpu.SemaphoreType.DMA,
        ],
    )
    (deg,) = f(dst3d)
    return deg


def _sc_aggregate(xs_flat, src3, dst_flat, npad):
    nch = src3.shape[1]
    rpt = npad // NS
    etile = nch * CHUNK

    def body(xs_hbm, src_hbm, dst_hbm, agg_hbm,
             srcv, si_a, si_b, di_a, di_b, buf_a, buf_b, acc,
             gsem_a, gsem_b, dsem_a, dsem_b):
        c = lax.axis_index("c")
        s = lax.axis_index("s")
        off = c * npad
        base = s * etile
        pltpu.sync_copy(src_hbm.at[s], srcv)

        def fill_z(i, carry):
            for j in range(8):
                buf_a[i, pl.ds(16 * j, 16)] = jnp.zeros((16,), jnp.float32)
            return carry

        lax.fori_loop(0, ZROWS, fill_z, 0)
        row0 = s * rpt
        for k in range(rpt // ZROWS):
            pltpu.sync_copy(
                buf_a.at[pl.ds(0, ZROWS)], acc.at[pl.ds(row0 + k * ZROWS, ZROWS)]
            )
        plsc.subcore_barrier()

        def fetch_di(k, di, dsem):
            pltpu.async_copy(dst_hbm.at[pl.ds(base + k * CHUNK, CHUNK)], di, dsem)

        def wait_di(k, di, dsem):
            pltpu.make_async_copy(
                dst_hbm.at[pl.ds(base + k * CHUNK, CHUNK)], di, dsem
            ).wait()

        _copy_row(srcv, 0, si_a, off)
        pltpu.async_copy(xs_hbm.at[si_a], buf_a, gsem_a)
        fetch_di(0, di_a, dsem_a)

        def step(i, carry):
            k0 = 2 * i
            k1 = 2 * i + 1
            k2 = 2 * i + 2
            _copy_row(srcv, k1, si_b, off)
            pltpu.async_copy(xs_hbm.at[si_b], buf_b, gsem_b)
            fetch_di(k1, di_b, dsem_b)
            pltpu.make_async_copy(xs_hbm.at[si_a], buf_a, gsem_a).wait()
            wait_di(k0, di_a, dsem_a)
            pltpu.sync_copy(buf_a, acc.at[di_a], add=True)

            @pl.when(k2 < nch)
            def _():
                _copy_row(srcv, k2, si_a, off)
                pltpu.async_copy(xs_hbm.at[si_a], buf_a, gsem_a)
                fetch_di(k2, di_a, dsem_a)

            pltpu.make_async_copy(xs_hbm.at[si_b], buf_b, gsem_b).wait()
            wait_di(k1, di_b, dsem_b)
            pltpu.sync_copy(buf_b, acc.at[di_b], add=True)
            return carry

        lax.fori_loop(0, nch // 2, step, 0)
        plsc.subcore_barrier()
        pltpu.sync_copy(
            acc.at[pl.ds(row0, rpt)], agg_hbm.at[pl.ds(off + row0, rpt)]
        )

    f = pl.kernel(
        body,
        out_type=[jax.ShapeDtypeStruct((NC * npad, 128), jnp.float32)],
        mesh=_mesh(),
        scratch_types=[
            pltpu.VMEM((nch, CHUNK), jnp.int32),
            pltpu.VMEM((CHUNK,), jnp.int32),
            pltpu.VMEM((CHUNK,), jnp.int32),
            pltpu.VMEM((CHUNK,), jnp.int32),
            pltpu.VMEM((CHUNK,), jnp.int32),
            pltpu.VMEM((CHUNK, 128), jnp.float32),
            pltpu.VMEM((CHUNK, 128), jnp.float32),
            pltpu.VMEM_SHARED((npad, 128), jnp.float32),
            pltpu.SemaphoreType.DMA,
            pltpu.SemaphoreType.DMA,
            pltpu.SemaphoreType.DMA,
            pltpu.SemaphoreType.DMA,
        ],
    )
    (agg,) = f(xs_flat, src3, dst_flat)
    return agg


def _tc_prep(x_pad, deg3, npad):
    def body(x_ref, d_ref, r_ref, xs_ref):
        d3 = d_ref[...]
        d = d3[0, :, :1] + d3[1, :, :1]
        r = lax.rsqrt(jnp.maximum(d, 1.0))
        r_ref[...] = jnp.broadcast_to(r, (BN, 16))
        xs = x_ref[...] * r
        xs_ref[0] = xs[:, :128]
        xs_ref[1] = xs[:, 128:]

    grid = (npad // BN,)
    return pl.pallas_call(
        body,
        grid=grid,
        in_specs=[
            pl.BlockSpec((BN, 256), lambda i: (i, 0)),
            pl.BlockSpec((2, BN, 128), lambda i: (0, i, 0)),
        ],
        out_specs=[
            pl.BlockSpec((BN, 16), lambda i: (i, 0)),
            pl.BlockSpec((2, BN, 128), lambda i: (0, i, 0)),
        ],
        out_shape=[
            jax.ShapeDtypeStruct((npad, 16), jnp.float32),
            jax.ShapeDtypeStruct((2, npad, 128), jnp.float32),
        ],
    )(x_pad, deg3)


def _tc_layer(agg3, x_pad, r16, w, n, npad, emit_next):
    def body(a_ref, x_ref, r_ref, w_ref, o_ref, *next_refs):
        r1 = r_ref[..., :1]
        a3 = a_ref[...]
        agg = jnp.concatenate([a3[0], a3[1]], axis=1)
        hi = (1.0 - ALPHA) * (r1 * agg) + ALPHA * x_ref[...]
        o = jnp.maximum(jnp.dot(hi, w_ref[...], preferred_element_type=jnp.float32), 0.0)
        o_ref[...] = o
        if emit_next:
            xs = o * r1
            next_refs[0][0] = xs[:, :128]
            next_refs[0][1] = xs[:, 128:]

    grid = (npad // BN,)
    out_specs = [pl.BlockSpec((BN, 256), lambda i: (i, 0))]
    out_shape = [jax.ShapeDtypeStruct((n, 256), jnp.float32)]
    if emit_next:
        out_specs.append(pl.BlockSpec((2, BN, 128), lambda i: (0, i, 0)))
        out_shape.append(jax.ShapeDtypeStruct((2, npad, 128), jnp.float32))
    return pl.pallas_call(
        body,
        grid=grid,
        in_specs=[
            pl.BlockSpec((2, BN, 128), lambda i: (0, i, 0)),
            pl.BlockSpec((BN, 256), lambda i: (i, 0)),
            pl.BlockSpec((BN, 16), lambda i: (i, 0)),
            pl.BlockSpec((256, 256), lambda i: (0, 0)),
        ],
        out_specs=out_specs,
        out_shape=out_shape,
    )(agg3, x_pad, r16, w)


def kernel(drug_cell_pair_feature, edge_idx, W):
    x = drug_cell_pair_feature
    n, d = x.shape
    e = edge_idx.shape[1]
    assert d == 256

    npad = ((n + 1 + NS * ZROWS - 1) // (NS * ZROWS)) * (NS * ZROWS)
    epad = ((e + 2 * NS * CHUNK - 1) // (2 * NS * CHUNK)) * (2 * NS * CHUNK)
    nch = epad // (NS * CHUNK)

    src = edge_idx[0]
    dst = edge_idx[1]
    src_pad = jnp.concatenate([src, jnp.zeros((epad - e,), jnp.int32)])
    dst_pad = jnp.concatenate([dst, jnp.full((epad - e,), n, jnp.int32)])
    src3 = src_pad.reshape(NS, nch, CHUNK)
    dst3d = dst_pad.reshape(NC * NS, nch // NC, CHUNK)
    x_pad = jnp.concatenate([x, jnp.zeros((npad - n, d), jnp.float32)])

    deg = _sc_degree(dst3d, npad)
    r16, xs3 = _tc_prep(x_pad, deg.reshape(NC, npad, 128), npad)

    outs = []
    cur = xs3
    for layer in range(5):
        agg = _sc_aggregate(cur.reshape(NC * npad, 128), src3, dst_pad, npad)
        agg3 = agg.reshape(NC, npad, 128)
        if layer < 4:
            o, cur = _tc_layer(agg3, x_pad, r16, W, n, npad, True)
        else:
            (o,) = _tc_layer(agg3, x_pad, r16, W, n, npad, False)
        outs.append(o)
    return tuple(outs)
